# Initial kernel scaffold; baseline (speedup 1.0000x reference)
import functools

import jax
import jax.numpy as jnp
from jax import lax
from jax.experimental import pallas as pl
from jax.experimental.pallas import tpu as pltpu
from jax.experimental.pallas import tpu_sc as plsc

B = 16384
D = 128
L = 16
NC = 2
NS = 16
NW = NC * NS
BPW = B // NW
CH = 128
NCHUNK = BPW // CH


def _glove_body(wi_hbm, ci_hbm, wemb_hbm, cemb_hbm, wb_hbm, cb_hbm, out_hbm,
                wi_idx, ci_idx, wrows, crows, bi_v, bj_v, out_v,
                sem_w0, sem_w1, sem_c0, sem_c1, sem_b):
    wid = lax.axis_index("s") * NC + lax.axis_index("c")
    base = wid * BPW

    pltpu.sync_copy(wi_hbm.at[pl.ds(base, BPW)], wi_idx)
    pltpu.sync_copy(ci_hbm.at[pl.ds(base, BPW)], ci_idx)

    sem_w = (sem_w0, sem_w1)
    sem_c = (sem_c0, sem_c1)

    bias_copies = []
    for k in range(NCHUNK):
        s = pl.ds(k * CH, CH)
        bias_copies.append(
            pltpu.async_copy(wb_hbm.at[wi_idx.at[s]], bi_v.at[s], sem_b))
        bias_copies.append(
            pltpu.async_copy(cb_hbm.at[ci_idx.at[s]], bj_v.at[s], sem_b))

    def fire(k):
        buf = k % 2
        s = pl.ds(k * CH, CH)
        cw = pltpu.async_copy(wemb_hbm.at[wi_idx.at[s]], wrows.at[buf], sem_w[buf])
        cc = pltpu.async_copy(cemb_hbm.at[ci_idx.at[s]], crows.at[buf], sem_c[buf])
        return cw, cc

    def compute(k):
        buf = k % 2

        def group(g, carry):
            for i in range(L):
                r = g * L + i
                p = (wrows[buf, r, pl.ds(0, L)] *
                     crows[buf, r, pl.ds(0, L)])
                for dd in range(1, D // L):
                    p = p + (wrows[buf, r, pl.ds(dd * L, L)] *
                             crows[buf, r, pl.ds(dd * L, L)])
                out_v[k * CH + r] = jnp.sum(p)
            sl = pl.ds(k * CH + g * L, L)
            out_v[sl] = out_v[sl] + bi_v[sl] + bj_v[sl]
            return carry

        lax.fori_loop(0, CH // L, group, 0)

    for c in bias_copies:
        c.wait()

    copies = [None, None]
    copies[0] = fire(0)
    for k in range(NCHUNK):
        if k + 1 < NCHUNK:
            copies[(k + 1) % 2] = fire(k + 1)
        cw, cc = copies[k % 2]
        cw.wait()
        cc.wait()
        compute(k)

    pltpu.sync_copy(out_v, out_hbm.at[pl.ds(base, BPW)])


@functools.partial(
    pl.kernel,
    mesh=plsc.VectorSubcoreMesh(core_axis_name="c", subcore_axis_name="s"),
    out_type=jax.ShapeDtypeStruct((B,), jnp.float32),
    scratch_types=[
        pltpu.VMEM((BPW,), jnp.int32),
        pltpu.VMEM((BPW,), jnp.int32),
        pltpu.VMEM((2, CH, D), jnp.float32),
        pltpu.VMEM((2, CH, D), jnp.float32),
        pltpu.VMEM((BPW,), jnp.float32),
        pltpu.VMEM((BPW,), jnp.float32),
        pltpu.VMEM((BPW,), jnp.float32),
        pltpu.SemaphoreType.DMA,
        pltpu.SemaphoreType.DMA,
        pltpu.SemaphoreType.DMA,
        pltpu.SemaphoreType.DMA,
        pltpu.SemaphoreType.DMA,
    ],
)
def _glove_sc(*refs):
    _glove_body(*refs)


def kernel(word_indices, context_word_indices, word_embeddings,
           context_word_embeddings, word_bias, context_word_bias):
    wb = word_bias.reshape(-1)
    cb = context_word_bias.reshape(-1)
    return _glove_sc(word_indices, context_word_indices, word_embeddings,
                     context_word_embeddings, wb, cb)


# speedup vs baseline: 1.1362x; 1.1362x over previous
import functools

import jax
import jax.numpy as jnp
from jax import lax
from jax.experimental import pallas as pl
from jax.experimental.pallas import tpu as pltpu
from jax.experimental.pallas import tpu_sc as plsc

B = 16384
D = 128
L = 16
NC = 2
NS = 16
NW = NC * NS
BPW = B // NW
CH = 128
NCHUNK = BPW // CH


def _glove_body(wi_hbm, ci_hbm, wemb_hbm, cemb_hbm, wb_hbm, cb_hbm, out_hbm,
                wi_idx, ci_idx, wrows, crows, bi_v, bj_v, out_v,
                sem_w0, sem_w1, sem_c0, sem_c1, sem_b):
    wid = lax.axis_index("s") * NC + lax.axis_index("c")
    base = wid * BPW

    pltpu.sync_copy(wi_hbm.at[pl.ds(base, BPW)], wi_idx)
    pltpu.sync_copy(ci_hbm.at[pl.ds(base, BPW)], ci_idx)

    sem_w = (sem_w0, sem_w1)
    sem_c = (sem_c0, sem_c1)

    bias_copies = []
    for k in range(NCHUNK):
        s = pl.ds(k * CH, CH)
        bias_copies.append(
            pltpu.async_copy(wb_hbm.at[wi_idx.at[s]], bi_v.at[s], sem_b))
        bias_copies.append(
            pltpu.async_copy(cb_hbm.at[ci_idx.at[s]], bj_v.at[s], sem_b))

    def fire(k):
        buf = k % 2
        s = pl.ds(k * CH, CH)
        cw = pltpu.async_copy(wemb_hbm.at[wi_idx.at[s]], wrows.at[buf], sem_w[buf])
        cc = pltpu.async_copy(cemb_hbm.at[ci_idx.at[s]], crows.at[buf], sem_c[buf])
        return cw, cc

    iota = lax.iota(jnp.int32, L)
    perms = [(iota + (1 << s)) & (L - 1) for s in range(3, -1, -1)]
    masks = [iota == i for i in range(L)]
    dnums = lax.GatherDimensionNumbers(
        offset_dims=(), collapsed_slice_dims=(0,), start_index_map=(0,))

    def rot(v, perm):
        return lax.gather(v, perm[:, None], dimension_numbers=dnums,
                          slice_sizes=(1,),
                          mode=lax.GatherScatterMode.PROMISE_IN_BOUNDS)

    def compute(k):
        buf = k % 2

        def group(g, carry):
            acc = None
            for i in range(L):
                r = g * L + i
                p = (wrows[buf, r, pl.ds(0, L)] *
                     crows[buf, r, pl.ds(0, L)])
                for dd in range(1, D // L):
                    p = p + (wrows[buf, r, pl.ds(dd * L, L)] *
                             crows[buf, r, pl.ds(dd * L, L)])
                for perm in perms:
                    p = p + rot(p, perm)
                acc = p if acc is None else jnp.where(masks[i], p, acc)
            sl = pl.ds(k * CH + g * L, L)
            out_v[sl] = acc + bi_v[sl] + bj_v[sl]
            return carry

        lax.fori_loop(0, CH // L, group, 0)

    for c in bias_copies:
        c.wait()

    copies = [None, None]
    copies[0] = fire(0)
    for k in range(NCHUNK):
        if k + 1 < NCHUNK:
            copies[(k + 1) % 2] = fire(k + 1)
        cw, cc = copies[k % 2]
        cw.wait()
        cc.wait()
        compute(k)

    pltpu.sync_copy(out_v, out_hbm.at[pl.ds(base, BPW)])


@functools.partial(
    pl.kernel,
    mesh=plsc.VectorSubcoreMesh(core_axis_name="c", subcore_axis_name="s"),
    out_type=jax.ShapeDtypeStruct((B,), jnp.float32),
    scratch_types=[
        pltpu.VMEM((BPW,), jnp.int32),
        pltpu.VMEM((BPW,), jnp.int32),
        pltpu.VMEM((2, CH, D), jnp.float32),
        pltpu.VMEM((2, CH, D), jnp.float32),
        pltpu.VMEM((BPW,), jnp.float32),
        pltpu.VMEM((BPW,), jnp.float32),
        pltpu.VMEM((BPW,), jnp.float32),
        pltpu.SemaphoreType.DMA,
        pltpu.SemaphoreType.DMA,
        pltpu.SemaphoreType.DMA,
        pltpu.SemaphoreType.DMA,
        pltpu.SemaphoreType.DMA,
    ],
)
def _glove_sc(*refs):
    _glove_body(*refs)


def kernel(word_indices, context_word_indices, word_embeddings,
           context_word_embeddings, word_bias, context_word_bias):
    wb = word_bias.reshape(-1)
    cb = context_word_bias.reshape(-1)
    return _glove_sc(word_indices, context_word_indices, word_embeddings,
                     context_word_embeddings, wb, cb)
